# baseline (speedup 1.0000x reference)
import math

import jax
import jax.numpy as jnp
import ml_dtypes
import numpy as np
from jax import lax
from jax.experimental import pallas as pl
from jax.experimental.pallas import tpu as pltpu
from jax.experimental.pallas import tpu_sc as plsc

D_MODEL = 1024
POS_START = 4
POS_SIZE = 128
PITCH_START = 132
PITCH_SIZE = 32
BAR_ID = 2
DOC_ID = 1
BASE = 10000.0
D_POS = D_MODEL // 2
D_PITCH = D_MODEL - D_POS

_LANES = 16
_BT = 2048


def _sincos_table(max_len, d_model):
    div_term = np.exp(
        np.arange(0, d_model, 2, dtype=np.float32) * (-math.log(BASE) / d_model)
    ).astype(np.float32)
    pos = np.arange(max_len, dtype=np.float32)[:, None]
    angle = (pos * div_term[None, :]).astype(np.float32)
    tab = np.stack((np.sin(angle), np.cos(angle)), axis=-1).reshape(max_len, d_model)
    return tab.astype(ml_dtypes.bfloat16)


_TPOS = _sincos_table(POS_SIZE, D_POS)
_TPIT = _sincos_table(PITCH_SIZE, D_PITCH)


def _forward_fill(token_ids):
    B, T = token_ids.shape
    rows_per_core = B // 2
    chunks_per_row = 16 // rows_per_core
    chunk_t = T // chunks_per_row
    n_vecs = chunk_t // _LANES
    n_sub = chunk_t // 128
    vecs_per_sub = 128 // _LANES

    def body(tok_hbm, out_hbm, tok_v, out_v):
        c = lax.axis_index("c")
        s = lax.axis_index("s")
        row_in_core = s // chunks_per_row
        chunk = s % chunks_per_row
        row = c * rows_per_core + row_in_core
        base = chunk * chunk_t

        pltpu.sync_copy(tok_hbm.at[row], tok_v)

        def keys_at(i):
            t0 = i * _LANES
            tok = tok_v[pl.ds(t0, _LANES)]
            pos_tok = (tok >= POS_START) & (tok < POS_START + POS_SIZE)
            upd = pos_tok | (tok == BAR_ID) | (tok == DOC_ID)
            t = t0 + lax.iota(jnp.int32, _LANES)
            val = jnp.where(pos_tok, tok - POS_START, 0)
            key = jnp.where(upd, t * 256 + val, -1)
            return tok, pos_tok, key

        def pstep(i, run):
            _, _, key = keys_at(i)
            return jnp.maximum(run, key)

        run_v = out_v

        def make_chunk(cpre):
            def _():
                r = lax.fori_loop(
                    cpre * n_vecs,
                    (cpre + 1) * n_vecs,
                    pstep,
                    jnp.zeros((_LANES,), jnp.int32),
                    unroll=4,
                )
                run_v[0, pl.ds(0, _LANES)] = jnp.maximum(
                    run_v[0, pl.ds(0, _LANES)], r
                )

            return _

        run_v[0, pl.ds(0, _LANES)] = jnp.zeros((_LANES,), jnp.int32)
        for cpre in range(chunks_per_row - 1):
            pl.when(cpre < chunk)(make_chunk(cpre))
        prefix = jnp.max(run_v[0, pl.ds(0, _LANES)])

        def step(i, carry):
            tok, pos_tok, key = keys_at(chunk * n_vecs + i)
            pitch_tok = (tok >= PITCH_START) & (tok < PITCH_START + PITCH_SIZE)
            filled = jnp.maximum(plsc.cummax(key), carry)
            pitch_idx = jnp.where(pitch_tok, tok - PITCH_START, 0)
            code = (
                jnp.bitwise_and(filled, 255)
                | (pos_tok.astype(jnp.int32) << 8)
                | (pitch_tok.astype(jnp.int32) << 9)
                | (pitch_idx << 10)
            )
            out_v[i // vecs_per_sub, pl.ds((i % vecs_per_sub) * _LANES, _LANES)] = code
            return jnp.max(filled)

        lax.fori_loop(0, n_vecs, step, prefix)
        j = base // _BT
        r0 = pl.multiple_of((base % _BT) // 128, n_sub)
        pltpu.sync_copy(out_v, out_hbm.at[row, j, pl.ds(r0, n_sub)])

    mesh = plsc.VectorSubcoreMesh(core_axis_name="c", subcore_axis_name="s")
    return pl.kernel(
        body,
        out_type=jax.ShapeDtypeStruct((B, T // _BT, _BT // 128, 128), jnp.int32),
        mesh=mesh,
        compiler_params=pltpu.CompilerParams(needs_layout_passes=False),
        scratch_types=[
            pltpu.VMEM((T,), jnp.int32),
            pltpu.VMEM((n_sub, 128), jnp.int32),
        ],
    )(token_ids)


def _pe_add_body(code_ref, x_ref, tpos_ref, tpit_ref, out_ref):
    code = code_ref[0, 0]
    r_chunks = code.shape[0]
    code_t = jnp.transpose(code)
    pos_idx = code_t & 255
    pos_tok = (code_t >> 8) & 1
    pitch_tok = (code_t >> 9) & 1
    pitch_idx = code_t >> 10
    sqrt2 = jnp.float32(math.sqrt(D_MODEL)) / jnp.sqrt(jnp.float32(D_POS))
    a = jnp.where(
        pitch_tok == 1, 1.0, jnp.where(pos_tok == 1, sqrt2, 0.0)
    ).astype(jnp.float32)
    b = pitch_tok.astype(jnp.float32)
    iota_p = lax.broadcasted_iota(jnp.int32, (128, POS_SIZE), 1)
    iota_t = lax.broadcasted_iota(jnp.int32, (128, PITCH_SIZE), 1)
    wp_chunks = []
    wt_chunks = []
    for r in range(r_chunks):
        wp_chunks.append(
            jnp.where(pos_idx[:, r : r + 1] == iota_p, a[:, r : r + 1], 0.0)
        )
        wt_chunks.append(
            jnp.where(pitch_idx[:, r : r + 1] == iota_t, b[:, r : r + 1], 0.0)
        )
    w_pos = jnp.concatenate(wp_chunks, axis=0).astype(jnp.bfloat16)
    w_pit = jnp.concatenate(wt_chunks, axis=0).astype(jnp.bfloat16)
    pe_pos = jnp.dot(w_pos, tpos_ref[...], preferred_element_type=jnp.float32)
    pe_pit = jnp.dot(w_pit, tpit_ref[...], preferred_element_type=jnp.float32)
    out_ref[0, :, :D_POS] = x_ref[0, :, :D_POS] + pe_pos
    out_ref[0, :, D_POS:] = x_ref[0, :, D_POS:] + pe_pit


def _pe_add(code3, x, tpos, tpit, bt):
    B, nb, r_chunks, _ = code3.shape
    grid = (B, nb)
    return pl.pallas_call(
        _pe_add_body,
        grid=grid,
        in_specs=[
            pl.BlockSpec((1, 1, r_chunks, 128), lambda i, j: (i, j, 0, 0)),
            pl.BlockSpec((1, bt, D_MODEL), lambda i, j: (i, j, 0)),
            pl.BlockSpec((POS_SIZE, D_POS), lambda i, j: (0, 0)),
            pl.BlockSpec((PITCH_SIZE, D_PITCH), lambda i, j: (0, 0)),
        ],
        out_specs=pl.BlockSpec((1, bt, D_MODEL), lambda i, j: (i, j, 0)),
        out_shape=jax.ShapeDtypeStruct(x.shape, x.dtype),
    )(code3, x, tpos, tpit)


def kernel(token_ids, x):
    code3 = _forward_fill(token_ids)
    return _pe_add(code3, x, _TPOS, _TPIT, _BT)


# speedup vs baseline: 1.0001x; 1.0001x over previous
import math

import jax
import jax.numpy as jnp
import ml_dtypes
import numpy as np
from jax import lax
from jax.experimental import pallas as pl
from jax.experimental.pallas import tpu as pltpu
from jax.experimental.pallas import tpu_sc as plsc

D_MODEL = 1024
POS_START = 4
POS_SIZE = 128
PITCH_START = 132
PITCH_SIZE = 32
BAR_ID = 2
DOC_ID = 1
BASE = 10000.0
D_POS = D_MODEL // 2
D_PITCH = D_MODEL - D_POS

_LANES = 16
_BT = 2048


def _sincos_table(max_len, d_model):
    div_term = np.exp(
        np.arange(0, d_model, 2, dtype=np.float32) * (-math.log(BASE) / d_model)
    ).astype(np.float32)
    pos = np.arange(max_len, dtype=np.float32)[:, None]
    angle = (pos * div_term[None, :]).astype(np.float32)
    tab = np.stack((np.sin(angle), np.cos(angle)), axis=-1).reshape(max_len, d_model)
    return tab.astype(ml_dtypes.bfloat16)


_TPOS = _sincos_table(POS_SIZE, D_POS)
_TPIT = _sincos_table(PITCH_SIZE, D_PITCH)


def _forward_fill(token_ids):
    B, T = token_ids.shape
    rows_per_core = B // 2
    chunks_per_row = 16 // rows_per_core
    chunk_t = T // chunks_per_row
    n_vecs = chunk_t // _LANES
    n_sub = chunk_t // 128
    vecs_per_sub = 128 // _LANES

    def body(tok_hbm, out_hbm, tok_v, out_v):
        c = lax.axis_index("c")
        s = lax.axis_index("s")
        row_in_core = s // chunks_per_row
        chunk = s % chunks_per_row
        row = c * rows_per_core + row_in_core
        base = chunk * chunk_t

        pltpu.sync_copy(tok_hbm.at[row], tok_v)

        def keys_at(i):
            t0 = i * _LANES
            tok = tok_v[pl.ds(t0, _LANES)]
            pos_tok = (tok >= POS_START) & (tok < POS_START + POS_SIZE)
            upd = pos_tok | (tok == BAR_ID) | (tok == DOC_ID)
            t = t0 + lax.iota(jnp.int32, _LANES)
            val = jnp.where(pos_tok, tok - POS_START, 0)
            key = jnp.where(upd, t * 256 + val, -1)
            return tok, pos_tok, key

        def pstep(i, run):
            _, _, key = keys_at(i)
            return jnp.maximum(run, key)

        n_pre = chunk * n_vecs
        run = lax.fori_loop(
            0, n_pre, pstep, jnp.zeros((_LANES,), jnp.int32)
        )
        prefix = jnp.max(run)

        def step(i, carry):
            tok, pos_tok, key = keys_at(chunk * n_vecs + i)
            pitch_tok = (tok >= PITCH_START) & (tok < PITCH_START + PITCH_SIZE)
            filled = jnp.maximum(plsc.cummax(key), carry)
            pitch_idx = jnp.where(pitch_tok, tok - PITCH_START, 0)
            code = (
                jnp.bitwise_and(filled, 255)
                | (pos_tok.astype(jnp.int32) << 8)
                | (pitch_tok.astype(jnp.int32) << 9)
                | (pitch_idx << 10)
            )
            out_v[i // vecs_per_sub, pl.ds((i % vecs_per_sub) * _LANES, _LANES)] = code
            return jnp.max(filled)

        lax.fori_loop(0, n_vecs, step, prefix)
        j = base // _BT
        r0 = pl.multiple_of((base % _BT) // 128, n_sub)
        pltpu.sync_copy(out_v, out_hbm.at[row, j, pl.ds(r0, n_sub)])

    mesh = plsc.VectorSubcoreMesh(core_axis_name="c", subcore_axis_name="s")
    return pl.kernel(
        body,
        out_type=jax.ShapeDtypeStruct((B, T // _BT, _BT // 128, 128), jnp.int32),
        mesh=mesh,
        compiler_params=pltpu.CompilerParams(needs_layout_passes=False),
        scratch_types=[
            pltpu.VMEM((T,), jnp.int32),
            pltpu.VMEM((n_sub, 128), jnp.int32),
        ],
    )(token_ids)


def _pe_add_body(code_ref, x_ref, tpos_ref, tpit_ref, out_ref):
    code = code_ref[0, 0]
    r_chunks = code.shape[0]
    code_t = jnp.transpose(code)
    pos_idx = code_t & 255
    pos_tok = (code_t >> 8) & 1
    pitch_tok = (code_t >> 9) & 1
    pitch_idx = code_t >> 10
    sqrt2 = jnp.float32(math.sqrt(D_MODEL)) / jnp.sqrt(jnp.float32(D_POS))
    a = jnp.where(
        pitch_tok == 1, 1.0, jnp.where(pos_tok == 1, sqrt2, 0.0)
    ).astype(jnp.float32)
    b = pitch_tok.astype(jnp.float32)
    iota_p = lax.broadcasted_iota(jnp.int32, (128, POS_SIZE), 1)
    iota_t = lax.broadcasted_iota(jnp.int32, (128, PITCH_SIZE), 1)
    wp_chunks = []
    wt_chunks = []
    for r in range(r_chunks):
        wp_chunks.append(
            jnp.where(pos_idx[:, r : r + 1] == iota_p, a[:, r : r + 1], 0.0)
        )
        wt_chunks.append(
            jnp.where(pitch_idx[:, r : r + 1] == iota_t, b[:, r : r + 1], 0.0)
        )
    w_pos = jnp.concatenate(wp_chunks, axis=0).astype(jnp.bfloat16)
    w_pit = jnp.concatenate(wt_chunks, axis=0).astype(jnp.bfloat16)
    pe_pos = jnp.dot(w_pos, tpos_ref[...], preferred_element_type=jnp.float32)
    pe_pit = jnp.dot(w_pit, tpit_ref[...], preferred_element_type=jnp.float32)
    out_ref[0, :, :D_POS] = x_ref[0, :, :D_POS] + pe_pos
    out_ref[0, :, D_POS:] = x_ref[0, :, D_POS:] + pe_pit


def _pe_add(code3, x, tpos, tpit, bt):
    B, nb, r_chunks, _ = code3.shape
    grid = (B, nb)
    return pl.pallas_call(
        _pe_add_body,
        grid=grid,
        in_specs=[
            pl.BlockSpec((1, 1, r_chunks, 128), lambda i, j: (i, j, 0, 0)),
            pl.BlockSpec((1, bt, D_MODEL), lambda i, j: (i, j, 0)),
            pl.BlockSpec((POS_SIZE, D_POS), lambda i, j: (0, 0)),
            pl.BlockSpec((PITCH_SIZE, D_PITCH), lambda i, j: (0, 0)),
        ],
        out_specs=pl.BlockSpec((1, bt, D_MODEL), lambda i, j: (i, j, 0)),
        out_shape=jax.ShapeDtypeStruct(x.shape, x.dtype),
    )(code3, x, tpos, tpit)


def kernel(token_ids, x):
    code3 = _forward_fill(token_ids)
    return _pe_add(code3, x, _TPOS, _TPIT, _BT)
